# baseline (speedup 1.0000x reference)
import functools

import jax
import jax.numpy as jnp
from jax import lax
from jax.experimental import pallas as pl
from jax.experimental.pallas import tpu as pltpu
from jax.experimental.pallas import tpu_sc as plsc

_B, _L, _D = 4096, 200, 128
_NC, _NS = 2, 16
_NW = _NC * _NS
_BPW = _B // _NW
_K = 8


@functools.partial(
    pl.kernel,
    mesh=plsc.VectorSubcoreMesh(core_axis_name="c", subcore_axis_name="s"),
    out_type=jax.ShapeDtypeStruct((_B, _L, _D), jnp.float32),
    scratch_types=[
        pltpu.VMEM((_L, _D), jnp.float32),
        pltpu.SemaphoreType.DMA,
    ],
)
def _sc_broadcast(table_hbm, out_hbm, tab_v, sem):
    wid = lax.axis_index("s") * _NC + lax.axis_index("c")
    base = wid * _BPW
    pltpu.sync_copy(table_hbm.at[pl.ds(0, _L)], tab_v)

    def chunk(j, c):
        row = base + j * _K
        for t in range(_K):
            pltpu.make_async_copy(tab_v, out_hbm.at[row + t], sem).start()
        for t in range(_K):
            pltpu.make_async_copy(tab_v, out_hbm.at[row + t], sem).wait()
        return c

    lax.fori_loop(0, _BPW // _K, chunk, 0)


def kernel(sequence, table):
    return _sc_broadcast(table)
